# speedup vs baseline: 1.0544x; 1.0544x over previous
import functools

import jax
import jax.numpy as jnp
from jax import lax
from jax.experimental import pallas as pl
from jax.experimental.pallas import tpu as pltpu
from jax.experimental.pallas import tpu_sc as plsc

BATCH = 16384
N_ITEMS = 1000000

_NC = 2
_NS = 16
_NW = _NC * _NS
_LANES = 16

_B_PER_W = BATCH // _NW
_GATHER_CHUNK = 128
_N_CHUNKS = _B_PER_W // _GATHER_CHUNK


@functools.partial(
    pl.kernel,
    mesh=plsc.VectorSubcoreMesh(core_axis_name="c", subcore_axis_name="s"),
    out_type=jax.ShapeDtypeStruct((BATCH,), jnp.float32),
    scratch_types=[
        pltpu.VMEM((_B_PER_W,), jnp.int32),
        pltpu.VMEM((_B_PER_W,), jnp.float32),
        pltpu.SemaphoreType.DMA,
    ],
)
def _sc_lookup_sigmoid(idx_hbm, table_hbm, out_hbm, idx_v, rows_v, sem):
    wid = lax.axis_index("s") * _NC + lax.axis_index("c")
    base = wid * _B_PER_W

    pltpu.sync_copy(idx_hbm.at[pl.ds(base, _B_PER_W)], idx_v)

    copies = []
    for j in range(_N_CHUNKS):
        sl = pl.ds(j * _GATHER_CHUNK, _GATHER_CHUNK)
        copies.append(
            pltpu.async_copy(table_hbm.at[idx_v.at[sl]], rows_v.at[sl], sem)
        )
    for c in copies:
        c.wait()

    for i in range(_B_PER_W // _LANES):
        sl = pl.ds(i * _LANES, _LANES)
        x = rows_v[sl]
        rows_v[sl] = 1.0 / (1.0 + jnp.exp(-x))

    pltpu.sync_copy(rows_v, out_hbm.at[pl.ds(base, _B_PER_W)])


def kernel(user_ids, item_ids, item_emb_weight):
    del user_ids
    idx = item_ids.astype(jnp.int32)
    table = item_emb_weight.reshape((N_ITEMS,))
    return _sc_lookup_sigmoid(idx, table)


# speedup vs baseline: 1.0635x; 1.0086x over previous
import functools

import jax
import jax.numpy as jnp
from jax import lax
from jax.experimental import pallas as pl
from jax.experimental.pallas import tpu as pltpu
from jax.experimental.pallas import tpu_sc as plsc

BATCH = 16384
N_ITEMS = 1000000

_NS = 16
_LANES = 16

_B_PER_W = BATCH // _NS
_GATHER_CHUNK = 128
_N_CHUNKS = _B_PER_W // _GATHER_CHUNK


@functools.partial(
    pl.kernel,
    mesh=plsc.VectorSubcoreMesh(
        core_axis_name="c", subcore_axis_name="s", num_cores=1
    ),
    out_type=jax.ShapeDtypeStruct((BATCH,), jnp.float32),
    scratch_types=[
        pltpu.VMEM((_B_PER_W,), jnp.int32),
        pltpu.VMEM((_B_PER_W,), jnp.float32),
        pltpu.VMEM((_B_PER_W,), jnp.float32),
        pltpu.SemaphoreType.DMA((_N_CHUNKS,)),
        pltpu.SemaphoreType.DMA,
    ],
)
def _sc_lookup_sigmoid(idx_hbm, table_hbm, out_hbm, idx_v, rows_v, res_v,
                       gsems, osem):
    wid = lax.axis_index("s")
    base = wid * _B_PER_W

    pltpu.sync_copy(idx_hbm.at[pl.ds(base, _B_PER_W)], idx_v)

    gathers = []
    for j in range(_N_CHUNKS):
        sl = pl.ds(j * _GATHER_CHUNK, _GATHER_CHUNK)
        gathers.append(
            pltpu.async_copy(
                table_hbm.at[idx_v.at[sl]], rows_v.at[sl], gsems.at[j]
            )
        )

    outs = []
    for j in range(_N_CHUNKS):
        gathers[j].wait()
        for i in range(_GATHER_CHUNK // _LANES):
            sl = pl.ds(j * _GATHER_CHUNK + i * _LANES, _LANES)
            x = rows_v[sl]
            res_v[sl] = 1.0 / (1.0 + jnp.exp(-x))
        sl = pl.ds(j * _GATHER_CHUNK, _GATHER_CHUNK)
        outs.append(
            pltpu.async_copy(
                res_v.at[sl],
                out_hbm.at[pl.ds(base + j * _GATHER_CHUNK, _GATHER_CHUNK)],
                osem,
            )
        )
    for c in outs:
        c.wait()


def kernel(user_ids, item_ids, item_emb_weight):
    del user_ids
    idx = item_ids.astype(jnp.int32)
    table = item_emb_weight.reshape((N_ITEMS,))
    return _sc_lookup_sigmoid(idx, table)


# speedup vs baseline: 1.0650x; 1.0015x over previous
import functools

import jax
import jax.numpy as jnp
from jax import lax
from jax.experimental import pallas as pl
from jax.experimental.pallas import tpu as pltpu
from jax.experimental.pallas import tpu_sc as plsc

BATCH = 16384
N_ITEMS = 1000000

_NS = 16
_LANES = 16

_B_PER_W = BATCH // _NS
_CHUNK = 128
_N_CHUNKS = _B_PER_W // _CHUNK


@functools.partial(
    pl.kernel,
    mesh=plsc.VectorSubcoreMesh(
        core_axis_name="c", subcore_axis_name="s", num_cores=1
    ),
    out_type=jax.ShapeDtypeStruct((BATCH,), jnp.float32),
    scratch_types=[
        pltpu.VMEM((_B_PER_W,), jnp.int32),
        pltpu.VMEM((_B_PER_W,), jnp.float32),
        pltpu.VMEM((_B_PER_W,), jnp.float32),
        pltpu.SemaphoreType.DMA((_N_CHUNKS,)),
        pltpu.SemaphoreType.DMA((_N_CHUNKS,)),
        pltpu.SemaphoreType.DMA,
    ],
)
def _sc_lookup_sigmoid(idx_hbm, table_hbm, out_hbm, idx_v, rows_v, res_v,
                       isems, gsems, osem):
    wid = lax.axis_index("s")
    base = wid * _B_PER_W

    idx_copies = []
    for j in range(_N_CHUNKS):
        sl = pl.ds(j * _CHUNK, _CHUNK)
        idx_copies.append(
            pltpu.async_copy(
                idx_hbm.at[pl.ds(base + j * _CHUNK, _CHUNK)],
                idx_v.at[sl],
                isems.at[j],
            )
        )

    gathers = []
    for j in range(_N_CHUNKS):
        sl = pl.ds(j * _CHUNK, _CHUNK)
        idx_copies[j].wait()
        gathers.append(
            pltpu.async_copy(
                table_hbm.at[idx_v.at[sl]], rows_v.at[sl], gsems.at[j]
            )
        )

    outs = []
    for j in range(_N_CHUNKS):
        gathers[j].wait()
        for i in range(_CHUNK // _LANES):
            sl = pl.ds(j * _CHUNK + i * _LANES, _LANES)
            x = rows_v[sl]
            res_v[sl] = 1.0 / (1.0 + jnp.exp(-x))
        outs.append(
            pltpu.async_copy(
                res_v.at[pl.ds(j * _CHUNK, _CHUNK)],
                out_hbm.at[pl.ds(base + j * _CHUNK, _CHUNK)],
                osem,
            )
        )
    for c in outs:
        c.wait()


def kernel(user_ids, item_ids, item_emb_weight):
    del user_ids
    idx = item_ids.astype(jnp.int32)
    table = item_emb_weight.reshape((N_ITEMS,))
    return _sc_lookup_sigmoid(idx, table)
